# baseline (speedup 1.0000x reference)
import jax
import jax.numpy as jnp
from jax.experimental import pallas as pl
from jax.experimental.pallas import tpu as pltpu

N_LEADS = 12
D_MODEL = 256
BATCH = 16384
ROW = N_LEADS * D_MODEL
BLOCK_B = 1024


def _gather_body(pos_ref, w_ref, o_ref):
    for i in range(N_LEADS):
        o_ref[i, :] = w_ref[pos_ref[0, i], :]


def _add_body(enc_ref, x_ref, o_ref):
    o_ref[...] = x_ref[...] + enc_ref[...]


def kernel(x, encoding_weight, positions):
    pos2d = positions.astype(jnp.int32).reshape(1, N_LEADS)
    pos_enc = pl.pallas_call(
        _gather_body,
        in_specs=[
            pl.BlockSpec(memory_space=pltpu.SMEM),
            pl.BlockSpec(memory_space=pltpu.VMEM),
        ],
        out_shape=jax.ShapeDtypeStruct((N_LEADS, D_MODEL), jnp.float32),
    )(pos2d, encoding_weight)

    enc_row = pos_enc.reshape(1, ROW)
    x2 = x.reshape(BATCH, ROW)
    out = pl.pallas_call(
        _add_body,
        grid=(BATCH // BLOCK_B,),
        in_specs=[
            pl.BlockSpec((1, ROW), lambda i: (0, 0)),
            pl.BlockSpec((BLOCK_B, ROW), lambda i: (i, 0)),
        ],
        out_specs=pl.BlockSpec((BLOCK_B, ROW), lambda i: (i, 0)),
        out_shape=jax.ShapeDtypeStruct((BATCH, ROW), jnp.float32),
    )(enc_row, x2)
    return out.reshape(BATCH, N_LEADS, D_MODEL)


# speedup vs baseline: 1.4935x; 1.4935x over previous
import jax
import jax.numpy as jnp
from jax.experimental import pallas as pl
from jax.experimental.pallas import tpu as pltpu

N_LEADS = 12
D_MODEL = 256
BATCH = 16384
BLOCK_B = 512


def _gather_body(pos_ref, w_ref, o_ref):
    for i in range(N_LEADS):
        o_ref[i, :] = w_ref[pos_ref[0, i], :]


def _add_body(enc_ref, x_ref, o_ref):
    o_ref[...] = x_ref[...] + enc_ref[...][None, :, :]


def kernel(x, encoding_weight, positions):
    pos2d = positions.astype(jnp.int32).reshape(1, N_LEADS)
    pos_enc = pl.pallas_call(
        _gather_body,
        in_specs=[
            pl.BlockSpec(memory_space=pltpu.SMEM),
            pl.BlockSpec(memory_space=pltpu.VMEM),
        ],
        out_shape=jax.ShapeDtypeStruct((N_LEADS, D_MODEL), jnp.float32),
    )(pos2d, encoding_weight)

    return pl.pallas_call(
        _add_body,
        grid=(BATCH // BLOCK_B,),
        in_specs=[
            pl.BlockSpec((N_LEADS, D_MODEL), lambda i: (0, 0)),
            pl.BlockSpec((BLOCK_B, N_LEADS, D_MODEL), lambda i: (i, 0, 0)),
        ],
        out_specs=pl.BlockSpec((BLOCK_B, N_LEADS, D_MODEL), lambda i: (i, 0, 0)),
        out_shape=jax.ShapeDtypeStruct((BATCH, N_LEADS, D_MODEL), jnp.float32),
    )(pos_enc, x)
